# baseline (speedup 1.0000x reference)
import functools

import jax
from jax import lax
import jax.numpy as jnp
from jax.experimental import pallas as pl
from jax.experimental.pallas import tpu as pltpu

_BLK = 1024
_ROW = 128


def _tc_onehot(x3, di, n):
    nb = n // _BLK

    def body(x_ref, di_ref, o_ref):
        t = x_ref[0]
        ohT = (
            lax.broadcasted_iota(jnp.int32, (_ROW, _BLK), 0)
            == jnp.broadcast_to(t, (_ROW, _BLK))
        ).astype(jnp.bfloat16)
        o_ref[...] = lax.dot_general(
            ohT,
            di_ref[...].astype(jnp.bfloat16),
            (((0,), (0,)), ((), ())),
            preferred_element_type=jnp.float32,
        )

    return pl.pallas_call(
        body,
        grid=(nb,),
        in_specs=[
            pl.BlockSpec((1, 1, _BLK), lambda i: (i, 0, 0)),
            pl.BlockSpec((_ROW, _ROW), lambda i: (0, 0)),
        ],
        out_specs=pl.BlockSpec((_BLK, _ROW), lambda i: (i, 0)),
        out_shape=jax.ShapeDtypeStruct((n, _ROW), jnp.float32),
    )(x3, di)


def kernel(x, di):
    b, t = x.shape
    n = b * t
    x3 = x.reshape(n // _BLK, 1, _BLK)
    out = _tc_onehot(x3, di, n)
    return out.reshape(b, t, di.shape[1])


# speedup vs baseline: 2.6109x; 2.6109x over previous
import functools

import jax
from jax import lax
import jax.numpy as jnp
from jax.experimental import pallas as pl
from jax.experimental.pallas import tpu as pltpu
from jax.experimental.pallas import tpu_sc as plsc

_WINDOW = 128
_K = 2
_ROW = 128


def _gather_sc(idx3d, di, n):
    steps = n // (_K * _WINDOW)
    mesh = plsc.VectorSubcoreMesh(core_axis_name="c", subcore_axis_name="s")

    @functools.partial(
        pl.kernel,
        out_type=jax.ShapeDtypeStruct((n, _ROW), di.dtype),
        mesh=mesh,
        scratch_types=[
            pltpu.VMEM_SHARED((_ROW, _ROW), di.dtype),
            pltpu.SemaphoreType.DMA,
        ],
    )
    def run(table_hbm, i_hbm, o_hbm, table_shared, sem):
        pltpu.sync_copy(table_hbm, table_shared)

        def body(i_vmem, o_vmem):
            cps = [
                pltpu.async_copy(
                    table_shared.at[i_vmem.at[0, j]],
                    o_vmem.at[pl.ds(j * _WINDOW, _WINDOW)],
                    sem,
                )
                for j in range(_K)
            ]
            for cp in cps:
                cp.wait()

        pltpu.emit_pipeline(
            body,
            grid=(steps,),
            in_specs=[pl.BlockSpec((1, _K, _WINDOW), lambda i: (i, 0, 0))],
            out_specs=[pl.BlockSpec((_K * _WINDOW, _ROW), lambda i: (i, 0))],
            core_axis_name=("c", "s"),
            dimension_semantics=(pltpu.PARALLEL,),
        )(i_hbm, o_hbm)

    return run(di, idx3d)


def kernel(x, di):
    b, t = x.shape
    n = b * t
    idx3d = x.reshape(n // (_K * _WINDOW), _K, _WINDOW)
    out = _gather_sc(idx3d, di, n)
    return out.reshape(b, t, di.shape[1])


# speedup vs baseline: 2.6254x; 1.0056x over previous
import functools

import jax
from jax import lax
import jax.numpy as jnp
from jax.experimental import pallas as pl
from jax.experimental.pallas import tpu as pltpu
from jax.experimental.pallas import tpu_sc as plsc

_WINDOW = 128
_K = 2
_ROW = 128


def _gather_sc(idx3d, di, n):
    steps = n // (_K * _WINDOW)
    mesh = plsc.VectorSubcoreMesh(core_axis_name="c", subcore_axis_name="s")

    @functools.partial(
        pl.kernel,
        out_type=jax.ShapeDtypeStruct((n, _ROW), di.dtype),
        mesh=mesh,
        scratch_types=[
            pltpu.VMEM_SHARED((_K, _ROW, _ROW), di.dtype),
            pltpu.SemaphoreType.DMA,
        ],
    )
    def run(table_hbm, i_hbm, o_hbm, table_shared, sem):
        @pl.when(lax.axis_index("s") == 0)
        def _():
            for j in range(_K):
                pltpu.sync_copy(table_hbm, table_shared.at[j])

        plsc.subcore_barrier()

        def body(i_vmem, o_vmem):
            cps = [
                pltpu.async_copy(
                    table_shared.at[j].at[i_vmem.at[0, j]],
                    o_vmem.at[pl.ds(j * _WINDOW, _WINDOW)],
                    sem,
                )
                for j in range(_K)
            ]
            for cp in cps:
                cp.wait()

        pltpu.emit_pipeline(
            body,
            grid=(steps,),
            in_specs=[pl.BlockSpec((1, _K, _WINDOW), lambda i: (i, 0, 0))],
            out_specs=[pl.BlockSpec((_K * _WINDOW, _ROW), lambda i: (i, 0))],
            core_axis_name=("c", "s"),
            dimension_semantics=(pltpu.PARALLEL,),
        )(i_hbm, o_hbm)

    return run(di, idx3d)


def kernel(x, di):
    b, t = x.shape
    n = b * t
    idx3d = x.reshape(n // (_K * _WINDOW), _K, _WINDOW)
    out = _gather_sc(idx3d, di, n)
    return out.reshape(b, t, di.shape[1])


# speedup vs baseline: 2.6350x; 1.0036x over previous
import functools

import jax
from jax import lax
import jax.numpy as jnp
from jax.experimental import pallas as pl
from jax.experimental.pallas import tpu as pltpu
from jax.experimental.pallas import tpu_sc as plsc

_WINDOW = 128
_K = 2
_ROW = 128


def _gather_sc(idx3d, di, n):
    steps = n // (_K * _WINDOW)
    mesh = plsc.VectorSubcoreMesh(core_axis_name="c", subcore_axis_name="s")

    @functools.partial(
        pl.kernel,
        out_type=jax.ShapeDtypeStruct((n, _ROW), di.dtype),
        mesh=mesh,
        scratch_types=[
            pltpu.VMEM_SHARED((_ROW, _ROW), di.dtype),
            pltpu.SemaphoreType.DMA,
        ],
    )
    def run(table_hbm, i_hbm, o_hbm, table_shared, sem):
        @pl.when(lax.axis_index("s") == 0)
        def _():
            pltpu.sync_copy(table_hbm, table_shared)

        plsc.subcore_barrier()

        def body(i_vmem, o_vmem):
            cps = [
                pltpu.async_copy(
                    table_shared.at[i_vmem.at[0, j]],
                    o_vmem.at[pl.ds(j * _WINDOW, _WINDOW)],
                    sem,
                )
                for j in range(_K)
            ]
            for cp in cps:
                cp.wait()

        pltpu.emit_pipeline(
            body,
            grid=(steps,),
            in_specs=[pl.BlockSpec((1, _K, _WINDOW), lambda i: (i, 0, 0))],
            out_specs=[pl.BlockSpec((_K * _WINDOW, _ROW), lambda i: (i, 0))],
            core_axis_name=("c", "s"),
            dimension_semantics=(pltpu.PARALLEL,),
        )(i_hbm, o_hbm)

    return run(di, idx3d)


def kernel(x, di):
    b, t = x.shape
    n = b * t
    idx3d = x.reshape(n // (_K * _WINDOW), _K, _WINDOW)
    out = _gather_sc(idx3d, di, n)
    return out.reshape(b, t, di.shape[1])
